# baseline (speedup 1.0000x reference)
import functools

import jax
import jax.numpy as jnp
from jax import lax
from jax.experimental import pallas as pl
from jax.experimental.pallas import tpu as pltpu
from jax.experimental.pallas import tpu_sc as plsc

POOL = 8192
PLEN = 5
D = 768
TOPK = 8
BATCH = 1024
SEQ = 64
PROMPT_ROWS = TOPK * PLEN
TAIL = SEQ - PROMPT_ROWS

_NW = 32
_BPW = BATCH // _NW


def _l2norm(v):
    ss = jnp.sum(v ** 2, axis=1, keepdims=True)
    return v * lax.rsqrt(jnp.maximum(ss, 1e-12))


def _sim_body(x_ref, p_ref, o_ref):
    o_ref[...] = lax.dot_general(
        x_ref[...], p_ref[...],
        (((1,), (1,)), ((), ())),
        preferred_element_type=jnp.float32)


def _similarity(xn, pn):
    bt, pt = 256, 2048
    return pl.pallas_call(
        _sim_body,
        grid=(POOL // pt, BATCH // bt),
        in_specs=[pl.BlockSpec((bt, D), lambda j, i: (i, 0)),
                  pl.BlockSpec((pt, D), lambda j, i: (j, 0))],
        out_specs=pl.BlockSpec((bt, pt), lambda j, i: (i, j)),
        out_shape=jax.ShapeDtypeStruct((BATCH, POOL), jnp.float32),
    )(xn, pn)


def _topk_body(s_ref, idx_ref, psum_ref):
    bt = s_ref.shape[0]
    cols = lax.broadcasted_iota(jnp.int32, (bt, POOL), 1)
    cur = s_ref[...]
    ids = []
    tot = None
    for j in range(TOPK):
        m = jnp.max(cur, axis=1, keepdims=True)
        am = jnp.min(jnp.where(cur == m, cols, POOL), axis=1,
                     keepdims=True)
        ids.append(am)
        tot = m if tot is None else tot + m
        if j < TOPK - 1:
            cur = jnp.where(cols == am, -jnp.inf, cur)
    idx_ref[...] = jnp.concatenate(ids, axis=1)
    psum_ref[...] = jnp.sum(tot)[None, None, None]


def _topk(sim):
    bt = 256
    nt = BATCH // bt
    return pl.pallas_call(
        _topk_body,
        grid=(nt,),
        in_specs=[pl.BlockSpec((bt, POOL), lambda i: (i, 0))],
        out_specs=[pl.BlockSpec((bt, TOPK), lambda i: (i, 0)),
                   pl.BlockSpec((1, 1, 1), lambda i: (i, 0, 0))],
        out_shape=[jax.ShapeDtypeStruct((BATCH, TOPK), jnp.int32),
                   jax.ShapeDtypeStruct((nt, 1, 1), jnp.float32)],
    )(sim)


def _sc_assemble(xtail, prompt2, pn, idx40, idx8):
    mesh = plsc.VectorSubcoreMesh(core_axis_name="c", subcore_axis_name="s")

    @functools.partial(
        pl.kernel,
        out_type=[jax.ShapeDtypeStruct((BATCH, SEQ, D), jnp.float32),
                  jax.ShapeDtypeStruct((BATCH, TOPK, D), jnp.float32)],
        mesh=mesh,
        scratch_types=[pltpu.VMEM((_BPW, PROMPT_ROWS), jnp.int32),
                       pltpu.VMEM((_BPW, TOPK), jnp.int32),
                       pltpu.VMEM((PROMPT_ROWS, D), jnp.float32),
                       pltpu.VMEM((PROMPT_ROWS, D), jnp.float32),
                       pltpu.VMEM((TAIL, D), jnp.float32),
                       pltpu.VMEM((TAIL, D), jnp.float32),
                       pltpu.VMEM((TOPK, D), jnp.float32),
                       pltpu.VMEM((TOPK, D), jnp.float32),
                       pltpu.SemaphoreType.DMA,
                       pltpu.SemaphoreType.DMA],
    )
    def k(xt_hbm, prompt_hbm, pn_hbm, i40_hbm, i8_hbm, out_hbm, bkn_hbm,
          i40_v, i8_v, pbuf_a, pbuf_b, tbuf_a, tbuf_b, kbuf_a, kbuf_b,
          gsem_a, gsem_b):
        wid = lax.axis_index("c") * 16 + lax.axis_index("s")
        base = wid * _BPW
        pltpu.sync_copy(i40_hbm.at[pl.ds(base, _BPW)], i40_v)
        pltpu.sync_copy(i8_hbm.at[pl.ds(base, _BPW)], i8_v)

        def start(bl, pbuf, tbuf, kbuf, gsem):
            b = base + bl
            pltpu.async_copy(prompt_hbm.at[i40_v.at[bl]], pbuf, gsem)
            pltpu.async_copy(xt_hbm.at[b], tbuf, gsem)
            pltpu.async_copy(pn_hbm.at[i8_v.at[bl]], kbuf, gsem)

        def finish(bl, pbuf, tbuf, kbuf, gsem):
            b = base + bl
            pltpu.make_async_copy(
                prompt_hbm.at[pl.ds(0, PROMPT_ROWS)], pbuf, gsem).wait()
            pltpu.make_async_copy(xt_hbm.at[0], tbuf, gsem).wait()
            pltpu.make_async_copy(
                pn_hbm.at[pl.ds(0, TOPK)], kbuf, gsem).wait()
            pltpu.sync_copy(pbuf, out_hbm.at[b, pl.ds(0, PROMPT_ROWS)])
            pltpu.sync_copy(tbuf, out_hbm.at[b, pl.ds(PROMPT_ROWS, TAIL)])
            pltpu.sync_copy(kbuf, bkn_hbm.at[b])

        start(0, pbuf_a, tbuf_a, kbuf_a, gsem_a)

        @pl.loop(0, _BPW, step=2)
        def _(bl):
            start(bl + 1, pbuf_b, tbuf_b, kbuf_b, gsem_b)
            finish(bl, pbuf_a, tbuf_a, kbuf_a, gsem_a)

            @pl.when(bl + 2 < _BPW)
            def _():
                start(bl + 2, pbuf_a, tbuf_a, kbuf_a, gsem_a)

            finish(bl + 1, pbuf_b, tbuf_b, kbuf_b, gsem_b)

    return k(xtail, prompt2, pn, idx40, idx8)


def kernel(x_embed, prompt, prompt_key):
    xn = _l2norm(jnp.mean(x_embed, axis=1))
    pn = _l2norm(prompt_key)
    xtail = x_embed[:, PROMPT_ROWS:, :]
    sim = _similarity(xn, pn)
    idx, psums = _topk(sim)
    reduce_sim = jnp.sum(psums) / BATCH
    idx40 = (idx[:, :, None] * PLEN
             + jnp.arange(PLEN, dtype=jnp.int32)).reshape(BATCH, PROMPT_ROWS)
    prompt2 = prompt.reshape(POOL * PLEN, D)
    out, bkn = _sc_assemble(xtail, prompt2, pn, idx40, idx)
    return (out, reduce_sim, sim, idx, bkn)


# speedup vs baseline: 1.7479x; 1.7479x over previous
import functools

import jax
import jax.numpy as jnp
from jax import lax
from jax.experimental import pallas as pl
from jax.experimental.pallas import tpu as pltpu
from jax.experimental.pallas import tpu_sc as plsc

POOL = 8192
PLEN = 5
D = 768
TOPK = 8
BATCH = 1024
SEQ = 64
PROMPT_ROWS = TOPK * PLEN
TAIL = SEQ - PROMPT_ROWS

_NW = 32
_BPW = BATCH // _NW


def _l2norm(v):
    ss = jnp.sum(v ** 2, axis=1, keepdims=True)
    return v * lax.rsqrt(jnp.maximum(ss, 1e-12))


def _sim_body(x_ref, p_ref, o_ref):
    o_ref[...] = lax.dot_general(
        x_ref[...], p_ref[...],
        (((1,), (1,)), ((), ())),
        preferred_element_type=jnp.float32)


def _similarity(xn, pn):
    bt, pt = 256, 2048
    return pl.pallas_call(
        _sim_body,
        grid=(POOL // pt, BATCH // bt),
        in_specs=[pl.BlockSpec((bt, D), lambda j, i: (i, 0)),
                  pl.BlockSpec((pt, D), lambda j, i: (j, 0))],
        out_specs=pl.BlockSpec((bt, pt), lambda j, i: (i, j)),
        out_shape=jax.ShapeDtypeStruct((BATCH, POOL), jnp.float32),
    )(xn, pn)


def _topk_body(s_ref, idx_ref, psum_ref):
    bt = s_ref.shape[0]
    cols = lax.broadcasted_iota(jnp.int32, (bt, POOL), 1)
    cur = s_ref[...]
    ids = []
    tot = None
    for j in range(TOPK):
        m = jnp.max(cur, axis=1, keepdims=True)
        am = jnp.min(jnp.where(cur == m, cols, POOL), axis=1,
                     keepdims=True)
        ids.append(am)
        tot = m if tot is None else tot + m
        if j < TOPK - 1:
            cur = jnp.where(cols == am, -jnp.inf, cur)
    idx_ref[...] = jnp.concatenate(ids, axis=1)
    psum_ref[...] = jnp.sum(tot)[None, None, None]


def _topk(sim):
    bt = 256
    nt = BATCH // bt
    return pl.pallas_call(
        _topk_body,
        grid=(nt,),
        in_specs=[pl.BlockSpec((bt, POOL), lambda i: (i, 0))],
        out_specs=[pl.BlockSpec((bt, TOPK), lambda i: (i, 0)),
                   pl.BlockSpec((1, 1, 1), lambda i: (i, 0, 0))],
        out_shape=[jax.ShapeDtypeStruct((BATCH, TOPK), jnp.int32),
                   jax.ShapeDtypeStruct((nt, 1, 1), jnp.float32)],
    )(sim)


def _sc_assemble(x, prompt2, pn, idx40, idx8):
    mesh = plsc.VectorSubcoreMesh(core_axis_name="c", subcore_axis_name="s")

    @functools.partial(
        pl.kernel,
        out_type=[jax.ShapeDtypeStruct((BATCH, SEQ, D), jnp.float32),
                  jax.ShapeDtypeStruct((BATCH, TOPK, D), jnp.float32)],
        mesh=mesh,
        scratch_types=[pltpu.VMEM((_BPW, PROMPT_ROWS), jnp.int32),
                       pltpu.VMEM((_BPW, TOPK), jnp.int32),
                       pltpu.VMEM((PROMPT_ROWS, D), jnp.float32),
                       pltpu.VMEM((PROMPT_ROWS, D), jnp.float32),
                       pltpu.VMEM((TAIL, D), jnp.float32),
                       pltpu.VMEM((TAIL, D), jnp.float32),
                       pltpu.VMEM((TOPK, D), jnp.float32),
                       pltpu.VMEM((TOPK, D), jnp.float32),
                       pltpu.SemaphoreType.DMA,
                       pltpu.SemaphoreType.DMA],
    )
    def k(x_hbm, prompt_hbm, pn_hbm, i40_hbm, i8_hbm, out_hbm, bkn_hbm,
          i40_v, i8_v, pbuf_a, pbuf_b, tbuf_a, tbuf_b, kbuf_a, kbuf_b,
          gsem_a, gsem_b):
        wid = lax.axis_index("c") * 16 + lax.axis_index("s")
        base = wid * _BPW
        pltpu.sync_copy(i40_hbm.at[pl.ds(base, _BPW)], i40_v)
        pltpu.sync_copy(i8_hbm.at[pl.ds(base, _BPW)], i8_v)

        def start(bl, pbuf, tbuf, kbuf, gsem):
            b = base + bl
            pltpu.async_copy(prompt_hbm.at[i40_v.at[bl]], pbuf, gsem)
            pltpu.async_copy(x_hbm.at[b, pl.ds(PROMPT_ROWS, TAIL)],
                             tbuf, gsem)
            pltpu.async_copy(pn_hbm.at[i8_v.at[bl]], kbuf, gsem)

        def finish(bl, pbuf, tbuf, kbuf, gsem):
            b = base + bl
            pltpu.make_async_copy(
                prompt_hbm.at[pl.ds(0, PROMPT_ROWS)], pbuf, gsem).wait()
            pltpu.make_async_copy(
                x_hbm.at[0, pl.ds(PROMPT_ROWS, TAIL)], tbuf, gsem).wait()
            pltpu.make_async_copy(
                pn_hbm.at[pl.ds(0, TOPK)], kbuf, gsem).wait()
            pltpu.sync_copy(pbuf, out_hbm.at[b, pl.ds(0, PROMPT_ROWS)])
            pltpu.sync_copy(tbuf, out_hbm.at[b, pl.ds(PROMPT_ROWS, TAIL)])
            pltpu.sync_copy(kbuf, bkn_hbm.at[b])

        start(0, pbuf_a, tbuf_a, kbuf_a, gsem_a)

        @pl.loop(0, _BPW, step=2)
        def _(bl):
            start(bl + 1, pbuf_b, tbuf_b, kbuf_b, gsem_b)
            finish(bl, pbuf_a, tbuf_a, kbuf_a, gsem_a)

            @pl.when(bl + 2 < _BPW)
            def _():
                start(bl + 2, pbuf_a, tbuf_a, kbuf_a, gsem_a)

            finish(bl + 1, pbuf_b, tbuf_b, kbuf_b, gsem_b)

    return k(x, prompt2, pn, idx40, idx8)


def kernel(x_embed, prompt, prompt_key):
    xn = _l2norm(jnp.mean(x_embed, axis=1))
    pn = _l2norm(prompt_key)
    sim = _similarity(xn, pn)
    idx, psums = _topk(sim)
    reduce_sim = jnp.sum(psums) / BATCH
    idx40 = (idx[:, :, None]
             + POOL * jnp.arange(PLEN, dtype=jnp.int32)).reshape(
                 BATCH, PROMPT_ROWS)
    prompt2 = jnp.transpose(prompt, (1, 0, 2)).reshape(PLEN * POOL, D)
    out, bkn = _sc_assemble(x_embed, prompt2, pn, idx40, idx)
    return (out, reduce_sim, sim, idx, bkn)
